# baseline (speedup 1.0000x reference)
import functools

import jax
import jax.numpy as jnp
from jax import lax
from jax.experimental import pallas as pl
from jax.experimental.pallas import tpu as pltpu
from jax.experimental.pallas import tpu_sc as plsc

_N = 200
_D = 64
_B = 4096
_NC = 2
_NS = 16
_NW = _NC * _NS
_BBLK = 512
_NBLK = _B // _BBLK
_TILES = _N * _NBLK
_TPW = _TILES // _NW
_LANES = _D // 16


def _emb_body(xT_hbm, item_hbm, pos_hbm, out_hbm, pos_v, idx_v, rows_v, sem):
    wid = lax.axis_index("s") * _NC + lax.axis_index("c")
    pltpu.sync_copy(pos_hbm, pos_v)
    t0 = wid * _TPW

    def tile_body(t, carry):
        tid = t0 + t
        n = tid // _NBLK
        b0 = (tid % _NBLK) * _BBLK
        pltpu.sync_copy(xT_hbm.at[n, pl.ds(b0, _BBLK)], idx_v)
        pltpu.async_copy(item_hbm.at[idx_v], rows_v, sem).wait()
        ps = [pos_v[n, pl.ds(16 * k, 16)] for k in range(_LANES)]

        @plsc.parallel_loop(0, _BBLK, 1, unroll=4)
        def row_add(r):
            for k in range(_LANES):
                rows_v[r, pl.ds(16 * k, 16)] = (
                    rows_v[r, pl.ds(16 * k, 16)] + ps[k]
                )

        pltpu.sync_copy(
            rows_v, out_hbm.at[pl.ds(b0, _BBLK), pl.ds(n * _D, _D)]
        )
        return carry

    lax.fori_loop(0, _TPW, tile_body, 0)


@jax.jit
def _emb_call(xT, item_table, pos_table):
    mesh = plsc.VectorSubcoreMesh(
        core_axis_name="c", subcore_axis_name="s"
    )
    run = pl.kernel(
        _emb_body,
        out_type=jax.ShapeDtypeStruct((_B, _N * _D), jnp.float32),
        mesh=mesh,
        compiler_params=pltpu.CompilerParams(use_tc_tiling_on_sc=False),
        scratch_types=[
            pltpu.VMEM((_N, _D), jnp.float32),
            pltpu.VMEM((_BBLK,), jnp.int32),
            pltpu.VMEM((_BBLK, _D), jnp.float32),
            pltpu.SemaphoreType.DMA,
        ],
    )
    return run(xT, item_table, pos_table)


def kernel(x, item_table, pos_table):
    xT = jnp.transpose(x.astype(jnp.int32))
    out = _emb_call(xT, item_table, pos_table)
    return out.reshape(_B, _N, _D)
